# baseline (speedup 1.0000x reference)
import functools

import jax
import jax.numpy as jnp
from jax import lax
from jax.experimental import pallas as pl
from jax.experimental.pallas import tpu as pltpu
from jax.experimental.pallas import tpu_sc as plsc


_BC = 1024
_SC_COLS = 4096
_CPW = 128
_RCH = 200
_C = 1000
_NCH = _C // _RCH
_NG = _CPW // 16

_mesh = plsc.VectorSubcoreMesh(core_axis_name="c", subcore_axis_name="s")


@functools.partial(
    pl.kernel,
    out_type=jax.ShapeDtypeStruct((_SC_COLS,), jnp.float32),
    mesh=_mesh,
    scratch_types=[
        pltpu.VMEM((_RCH, _CPW), jnp.float32),
        pltpu.VMEM((_RCH, _CPW), jnp.float32),
        pltpu.VMEM((_RCH, _CPW), jnp.float32),
        pltpu.VMEM((_RCH, _CPW), jnp.float32),
        pltpu.VMEM((_CPW,), jnp.float32),
        pltpu.SemaphoreType.DMA,
        pltpu.SemaphoreType.DMA,
        pltpu.SemaphoreType.DMA,
        pltpu.SemaphoreType.DMA,
    ],
)
def _sc_max(x_hbm, t_hbm, out_hbm, xb0, xb1, tb0, tb1, acc,
            sx0, sx1, st0, st1):
    wid = lax.axis_index("s") * 2 + lax.axis_index("c")
    base = wid * _CPW
    xbufs, tbufs = (xb0, xb1), (tb0, tb1)
    sxs, sts = (sx0, sx1), (st0, st1)

    def fire(ci, b):
        pltpu.async_copy(
            x_hbm.at[pl.ds(ci * _RCH, _RCH), pl.ds(base, _CPW)],
            xbufs[b], sxs[b])
        pltpu.async_copy(
            t_hbm.at[pl.ds(ci * _RCH, _RCH), pl.ds(base, _CPW)],
            tbufs[b], sts[b])

    fire(0, 0)
    fire(1, 1)

    def process(b, accs):
        xbuf, tbuf = xbufs[b], tbufs[b]

        def row8(r8, a):
            cur = a
            for dr in range(8):
                new = []
                for g in range(_NG):
                    x = xbuf[r8 * 8 + dr, pl.ds(g * 16, 16)]
                    t = tbuf[r8 * 8 + dr, pl.ds(g * 16, 16)]
                    new.append(jnp.maximum(
                        cur[g], jnp.where(t > 0.0, x, -jnp.inf)))
                cur = tuple(new)
            return cur

        return lax.fori_loop(0, _RCH // 8, row8, accs)

    def wait(b):
        pltpu.make_async_copy(
            x_hbm.at[pl.ds(0, _RCH), pl.ds(base, _CPW)], xbufs[b],
            sxs[b]).wait()
        pltpu.make_async_copy(
            t_hbm.at[pl.ds(0, _RCH), pl.ds(base, _CPW)], tbufs[b],
            sts[b]).wait()

    def outer(ci2, accs):
        for b in range(2):
            wait(b)
            accs = process(b, accs)

            @pl.when(2 * ci2 + b + 2 < _NCH)
            def _():
                fire(2 * ci2 + b + 2, b)

        return accs

    init = tuple(jnp.full((16,), -jnp.inf, jnp.float32) for _ in range(_NG))
    accs = lax.fori_loop(0, _NCH // 2, outer, init)
    wait(0)
    accs = process(0, accs)
    for g in range(_NG):
        acc[pl.ds(g * 16, 16)] = accs[g]
    pltpu.sync_copy(acc, out_hbm.at[pl.ds(base, _CPW)])


def _tc_body(x_ref, t_ref, out_ref):
    i = pl.program_id(0)
    x = x_ref[...]
    t = t_ref[...]
    masked = jnp.where(t > 0.0, x, -jnp.inf)
    m = jnp.max(masked, axis=0, keepdims=True)
    hp = jnp.max(t, axis=0, keepdims=True) > 0.0
    sig = jnp.clip(jax.nn.sigmoid(m), 1e-6, 1.0 - 1e-6)
    li = jnp.where(hp, -jnp.log(sig), 0.0)
    part = jnp.sum(li, axis=(0, 1), keepdims=True)

    @pl.when(i == 0)
    def _():
        out_ref[...] = jnp.zeros_like(out_ref)

    out_ref[...] += part


@jax.jit
def kernel(input, target):
    B, C = input.shape
    xT = input.T
    tT = target.T
    off = _SC_COLS // _BC
    nb = (B - _SC_COLS) // _BC

    sc_max = _sc_max(xT, tT)

    tc_part = pl.pallas_call(
        _tc_body,
        grid=(nb,),
        in_specs=[
            pl.BlockSpec((C, _BC), lambda i: (0, i + off)),
            pl.BlockSpec((C, _BC), lambda i: (0, i + off)),
        ],
        out_specs=pl.BlockSpec((1, 1), lambda i: (0, 0)),
        out_shape=jax.ShapeDtypeStruct((1, 1), jnp.float32),
    )(xT, tT)

    sig = jnp.clip(jax.nn.sigmoid(sc_max), 1e-6, 1.0 - 1e-6)
    sc_li = jnp.where(sc_max == -jnp.inf, 0.0, -jnp.log(sig))
    return (tc_part[0, 0] + jnp.sum(sc_li)) / B


# speedup vs baseline: 1.5029x; 1.5029x over previous
import jax
import jax.numpy as jnp
from jax.experimental import pallas as pl
from jax.experimental.pallas import tpu as pltpu


_BC = 1024


def _body(x_ref, t_ref, out_ref):
    i = pl.program_id(0)
    nb = pl.num_programs(0)
    x = x_ref[...]
    t = t_ref[...]
    masked = jnp.where(t > 0.0, x, -jnp.inf)
    m = jnp.max(masked, axis=0, keepdims=True)
    hp = jnp.max(t, axis=0, keepdims=True) > 0.0
    sig = jnp.clip(jax.nn.sigmoid(m), 1e-6, 1.0 - 1e-6)
    li = jnp.where(hp, -jnp.log(sig), 0.0)
    part = jnp.sum(li, axis=(0, 1), keepdims=True)

    @pl.when(i == 0)
    def _():
        out_ref[...] = jnp.zeros_like(out_ref)

    out_ref[...] += part

    @pl.when(i == nb - 1)
    def _():
        out_ref[...] = out_ref[...] * (1.0 / (nb * _BC))


@jax.jit
def kernel(input, target):
    B, C = input.shape
    xT = input.T
    tT = target.T
    nb = B // _BC
    out = pl.pallas_call(
        _body,
        grid=(nb,),
        in_specs=[
            pl.BlockSpec((C, _BC), lambda i: (0, i)),
            pl.BlockSpec((C, _BC), lambda i: (0, i)),
        ],
        out_specs=pl.BlockSpec((1, 1), lambda i: (0, 0)),
        out_shape=jax.ShapeDtypeStruct((1, 1), jnp.float32),
    )(xT, tT)
    return out[0, 0]
